# Initial kernel scaffold; baseline (speedup 1.0000x reference)
import functools

import jax
import jax.numpy as jnp
from jax import lax
from jax.experimental import pallas as pl
from jax.experimental.pallas import tpu as pltpu
from jax.experimental.pallas import tpu_sc as plsc

N = 10000
EH = 160000
E = 2 * EH
D = 128

NC = 2
NS = 16
NW = NC * NS
CH = 128
NCHUNK = E // CH
ITERS = (NCHUNK + NW - 1) // NW
RPT = N // NS

_mesh = plsc.VectorSubcoreMesh(core_axis_name="c", subcore_axis_name="s")


def _sc_scatter_add(h2, idx, zeros):
    @functools.partial(
        pl.kernel,
        out_type=jax.ShapeDtypeStruct((NC, N, D), jnp.float32),
        mesh=_mesh,
        scratch_types=[
            pltpu.VMEM((CH,), jnp.int32),
            pltpu.VMEM((CH, D), jnp.float32),
            pltpu.VMEM_SHARED((N, D), jnp.float32),
        ],
    )
    def k(h_hbm, idx_hbm, z_hbm, out_hbm, idx_v, rows_v, acc):
        c = lax.axis_index("c")
        s = lax.axis_index("s")
        wid = s * NC + c
        pltpu.sync_copy(z_hbm.at[pl.ds(s * RPT, RPT)], acc.at[pl.ds(s * RPT, RPT)])
        plsc.subcore_barrier()

        @pl.loop(0, ITERS)
        def _(i):
            g = i * NW + wid

            @pl.when(g < NCHUNK)
            def _():
                base = g * CH
                pltpu.sync_copy(idx_hbm.at[pl.ds(base, CH)], idx_v)
                pltpu.sync_copy(h_hbm.at[pl.ds(base, CH)], rows_v)
                pltpu.sync_copy(rows_v, acc.at[idx_v], add=True)

        plsc.subcore_barrier()
        pltpu.sync_copy(acc.at[pl.ds(s * RPT, RPT)],
                        out_hbm.at[c, pl.ds(s * RPT, RPT)])

    return k(h2.reshape(E, D), idx, zeros)


def _sc_gather(table, idx):
    @functools.partial(
        pl.kernel,
        out_type=jax.ShapeDtypeStruct((E, D), jnp.float32),
        mesh=_mesh,
        scratch_types=[
            pltpu.VMEM((CH,), jnp.int32),
            pltpu.VMEM((CH, D), jnp.float32),
        ],
    )
    def k(t_hbm, idx_hbm, out_hbm, idx_v, rows_v):
        c = lax.axis_index("c")
        s = lax.axis_index("s")
        wid = s * NC + c

        @pl.loop(0, ITERS)
        def _(i):
            g = i * NW + wid

            @pl.when(g < NCHUNK)
            def _():
                base = g * CH
                pltpu.sync_copy(idx_hbm.at[pl.ds(base, CH)], idx_v)
                pltpu.sync_copy(t_hbm.at[idx_v], rows_v)
                pltpu.sync_copy(rows_v, out_hbm.at[pl.ds(base, CH)])

    return k(table, idx)


def _matmul_body(x_ref, w_ref, o_ref):
    o_ref[...] = jnp.dot(x_ref[...], w_ref[...],
                         preferred_element_type=jnp.float32)


def _tc_matmul(x, w, bm=2000):
    m, kdim = x.shape
    dout = w.shape[1]
    return pl.pallas_call(
        _matmul_body,
        grid=(m // bm,),
        in_specs=[
            pl.BlockSpec((bm, kdim), lambda i: (i, 0)),
            pl.BlockSpec((kdim, dout), lambda i: (0, 0)),
        ],
        out_specs=pl.BlockSpec((bm, dout), lambda i: (i, 0)),
        out_shape=jax.ShapeDtypeStruct((m, dout), jnp.float32),
    )(x, w)


def _lin1_body(ea_ref, g_ref, w_ref, b_ref, o_ref):
    w = w_ref[...]
    b = b_ref[...]
    for p in range(2):
        o_ref[p] = jnp.maximum(
            g_ref[p] + jnp.dot(ea_ref[p], w, preferred_element_type=jnp.float32)
            + b, 0.0)


def _tc_lin1(ea2, g2, w1e, b1, bp=2000):
    de = ea2.shape[-1]
    return pl.pallas_call(
        _lin1_body,
        grid=(EH // bp,),
        in_specs=[
            pl.BlockSpec((2, bp, de), lambda i: (0, i, 0)),
            pl.BlockSpec((2, bp, D), lambda i: (0, i, 0)),
            pl.BlockSpec((de, D), lambda i: (0, 0)),
            pl.BlockSpec((1, D), lambda i: (0, 0)),
        ],
        out_specs=pl.BlockSpec((2, bp, D), lambda i: (0, i, 0)),
        out_shape=jax.ShapeDtypeStruct((2, EH, D), jnp.float32),
    )(ea2, g2, w1e, b1)


def _combine_body(h_ref, g_ref, w_ref, b_ref, o_ref):
    w = w_ref[...]
    b = b_ref[...]
    he = h_ref[0]
    ho = h_ref[1]
    o_ref[0] = jnp.maximum(
        he + jnp.dot(g_ref[0] - ho, w, preferred_element_type=jnp.float32) + b,
        0.0)
    o_ref[1] = jnp.maximum(
        ho + jnp.dot(g_ref[1] - he, w, preferred_element_type=jnp.float32) + b,
        0.0)


def _tc_combine(h2, g2, w, b, bp=2000):
    return pl.pallas_call(
        _combine_body,
        grid=(EH // bp,),
        in_specs=[
            pl.BlockSpec((2, bp, D), lambda i: (0, i, 0)),
            pl.BlockSpec((2, bp, D), lambda i: (0, i, 0)),
            pl.BlockSpec((D, D), lambda i: (0, 0)),
            pl.BlockSpec((1, D), lambda i: (0, 0)),
        ],
        out_specs=pl.BlockSpec((2, bp, D), lambda i: (0, i, 0)),
        out_shape=jax.ShapeDtypeStruct((2, EH, D), jnp.float32),
    )(h2, g2, w, b)


def _sum2_body(p_ref, o_ref):
    o_ref[...] = p_ref[0] + p_ref[1]


def _tc_sum2(parts, bn=2000):
    return pl.pallas_call(
        _sum2_body,
        grid=(N // bn,),
        in_specs=[pl.BlockSpec((2, bn, D), lambda i: (0, i, 0))],
        out_specs=pl.BlockSpec((bn, D), lambda i: (i, 0)),
        out_shape=jax.ShapeDtypeStruct((N, D), jnp.float32),
    )(parts)


def _final_body(x_ref, p_ref, wx_ref, wh_ref, b_ref, o_ref):
    agg = p_ref[0] + p_ref[1]
    o_ref[...] = jnp.maximum(
        jnp.dot(x_ref[...], wx_ref[...], preferred_element_type=jnp.float32)
        + jnp.dot(agg, wh_ref[...], preferred_element_type=jnp.float32)
        + b_ref[...], 0.0)


def _tc_final(x, parts, wax, wah, ba, bn=2000):
    return pl.pallas_call(
        _final_body,
        grid=(N // bn,),
        in_specs=[
            pl.BlockSpec((bn, D), lambda i: (i, 0)),
            pl.BlockSpec((2, bn, D), lambda i: (0, i, 0)),
            pl.BlockSpec((D, D), lambda i: (0, 0)),
            pl.BlockSpec((D, D), lambda i: (0, 0)),
            pl.BlockSpec((1, D), lambda i: (0, 0)),
        ],
        out_specs=pl.BlockSpec((bn, D), lambda i: (i, 0)),
        out_shape=jax.ShapeDtypeStruct((N, D), jnp.float32),
    )(x, parts, wax, wah, ba)


def kernel(x, edge_attr, W1, b1, Wm1, bm1, Wm2, bm2, Wm3, bm3, Wa, ba,
           edge_index):
    u = edge_index[0, 0::2].astype(jnp.int32)
    v = edge_index[1, 0::2].astype(jnp.int32)
    uv = jnp.concatenate([u, v])
    vu = jnp.concatenate([v, u])
    ea2 = jnp.stack([edge_attr[0::2], edge_attr[1::2]])
    zeros = jnp.zeros((N, D), jnp.float32)

    q = _tc_matmul(x, W1[:D])
    g2 = _sc_gather(q, uv).reshape(2, EH, D)
    h = _tc_lin1(ea2, g2, W1[D:], b1.reshape(1, D))

    for w, b in ((Wm1, bm1), (Wm2, bm2), (Wm3, bm3)):
        parts = _sc_scatter_add(h, vu, zeros)
        agg = _tc_sum2(parts)
        g2 = _sc_gather(agg, uv).reshape(2, EH, D)
        h = _tc_combine(h, g2, w, b.reshape(1, D))

    parts = _sc_scatter_add(h, vu, zeros)
    return _tc_final(x, parts, Wa[:D], Wa[D:], ba.reshape(1, D))


# speedup vs baseline: 2.3505x; 2.3505x over previous
import functools

import jax
import jax.numpy as jnp
from jax import lax
from jax.experimental import pallas as pl
from jax.experimental.pallas import tpu as pltpu
from jax.experimental.pallas import tpu_sc as plsc

N = 10000
EH = 160000
E = 2 * EH
D = 128

NC = 2
NS = 16
NW = NC * NS
CH = 128
NCHUNK = E // CH
ITERS = (NCHUNK + NW - 1) // NW
NP = 10240
RPT = NP // NS

_mesh = plsc.VectorSubcoreMesh(core_axis_name="c", subcore_axis_name="s")


def _sc_scatter_add(h2, idx, zeros):
    @functools.partial(
        pl.kernel,
        out_type=jax.ShapeDtypeStruct((NC, NP, D), jnp.float32),
        mesh=_mesh,
        scratch_types=[
            pltpu.VMEM((CH,), jnp.int32),
            pltpu.VMEM((CH, D), jnp.float32),
            pltpu.VMEM_SHARED((NP, D), jnp.float32),
        ],
    )
    def k(h_hbm, idx_hbm, z_hbm, out_hbm, idx_v, rows_v, acc):
        c = lax.axis_index("c")
        s = lax.axis_index("s")
        wid = s * NC + c
        pltpu.sync_copy(z_hbm.at[pl.ds(s * RPT, RPT)], acc.at[pl.ds(s * RPT, RPT)])
        plsc.subcore_barrier()

        @pl.loop(0, ITERS)
        def _(i):
            g = i * NW + wid

            @pl.when(g < NCHUNK)
            def _():
                base = g * CH
                pltpu.sync_copy(idx_hbm.at[pl.ds(base, CH)], idx_v)
                pltpu.sync_copy(h_hbm.at[pl.ds(base, CH)], rows_v)
                pltpu.sync_copy(rows_v, acc.at[idx_v], add=True)

        plsc.subcore_barrier()
        pltpu.sync_copy(acc.at[pl.ds(s * RPT, RPT)],
                        out_hbm.at[c, pl.ds(s * RPT, RPT)])

    return k(h2.reshape(E, D), idx, zeros)


def _sc_gather(table, idx):
    @functools.partial(
        pl.kernel,
        out_type=jax.ShapeDtypeStruct((E, D), jnp.float32),
        mesh=_mesh,
        scratch_types=[
            pltpu.VMEM((CH,), jnp.int32),
            pltpu.VMEM((CH, D), jnp.float32),
        ],
    )
    def k(t_hbm, idx_hbm, out_hbm, idx_v, rows_v):
        c = lax.axis_index("c")
        s = lax.axis_index("s")
        wid = s * NC + c

        @pl.loop(0, ITERS)
        def _(i):
            g = i * NW + wid

            @pl.when(g < NCHUNK)
            def _():
                base = g * CH
                pltpu.sync_copy(idx_hbm.at[pl.ds(base, CH)], idx_v)
                pltpu.sync_copy(t_hbm.at[idx_v], rows_v)
                pltpu.sync_copy(rows_v, out_hbm.at[pl.ds(base, CH)])

    return k(table, idx)


def _matmul_body(x_ref, w_ref, o_ref):
    o_ref[...] = jnp.dot(x_ref[...], w_ref[...],
                         preferred_element_type=jnp.float32)


def _tc_matmul(x, w, bm=2000):
    m, kdim = x.shape
    dout = w.shape[1]
    return pl.pallas_call(
        _matmul_body,
        grid=(m // bm,),
        in_specs=[
            pl.BlockSpec((bm, kdim), lambda i: (i, 0)),
            pl.BlockSpec((kdim, dout), lambda i: (0, 0)),
        ],
        out_specs=pl.BlockSpec((bm, dout), lambda i: (i, 0)),
        out_shape=jax.ShapeDtypeStruct((m, dout), jnp.float32),
    )(x, w)


def _lin1_body(ea_ref, g_ref, w_ref, b_ref, o_ref):
    w = w_ref[...]
    b = b_ref[...]
    for p in range(2):
        o_ref[p] = jnp.maximum(
            g_ref[p] + jnp.dot(ea_ref[p], w, preferred_element_type=jnp.float32)
            + b, 0.0)


def _tc_lin1(ea2, g2, w1e, b1, bp=2000):
    de = ea2.shape[-1]
    return pl.pallas_call(
        _lin1_body,
        grid=(EH // bp,),
        in_specs=[
            pl.BlockSpec((2, bp, de), lambda i: (0, i, 0)),
            pl.BlockSpec((2, bp, D), lambda i: (0, i, 0)),
            pl.BlockSpec((de, D), lambda i: (0, 0)),
            pl.BlockSpec((1, D), lambda i: (0, 0)),
        ],
        out_specs=pl.BlockSpec((2, bp, D), lambda i: (0, i, 0)),
        out_shape=jax.ShapeDtypeStruct((2, EH, D), jnp.float32),
    )(ea2, g2, w1e, b1)


def _combine_body(h_ref, g_ref, w_ref, b_ref, o_ref):
    w = w_ref[...]
    b = b_ref[...]
    he = h_ref[0]
    ho = h_ref[1]
    o_ref[0] = jnp.maximum(
        he + jnp.dot(g_ref[0] - ho, w, preferred_element_type=jnp.float32) + b,
        0.0)
    o_ref[1] = jnp.maximum(
        ho + jnp.dot(g_ref[1] - he, w, preferred_element_type=jnp.float32) + b,
        0.0)


def _tc_combine(h2, g2, w, b, bp=2000):
    return pl.pallas_call(
        _combine_body,
        grid=(EH // bp,),
        in_specs=[
            pl.BlockSpec((2, bp, D), lambda i: (0, i, 0)),
            pl.BlockSpec((2, bp, D), lambda i: (0, i, 0)),
            pl.BlockSpec((D, D), lambda i: (0, 0)),
            pl.BlockSpec((1, D), lambda i: (0, 0)),
        ],
        out_specs=pl.BlockSpec((2, bp, D), lambda i: (0, i, 0)),
        out_shape=jax.ShapeDtypeStruct((2, EH, D), jnp.float32),
    )(h2, g2, w, b)


def _sum2_body(p_ref, o_ref):
    o_ref[...] = p_ref[0] + p_ref[1]


def _tc_sum2(parts, bn=2000):
    return pl.pallas_call(
        _sum2_body,
        grid=(N // bn,),
        in_specs=[pl.BlockSpec((2, bn, D), lambda i: (0, i, 0))],
        out_specs=pl.BlockSpec((bn, D), lambda i: (i, 0)),
        out_shape=jax.ShapeDtypeStruct((N, D), jnp.float32),
    )(parts)


def _final_body(x_ref, p_ref, wx_ref, wh_ref, b_ref, o_ref):
    agg = p_ref[0] + p_ref[1]
    o_ref[...] = jnp.maximum(
        jnp.dot(x_ref[...], wx_ref[...], preferred_element_type=jnp.float32)
        + jnp.dot(agg, wh_ref[...], preferred_element_type=jnp.float32)
        + b_ref[...], 0.0)


def _tc_final(x, parts, wax, wah, ba, bn=2000):
    return pl.pallas_call(
        _final_body,
        grid=(N // bn,),
        in_specs=[
            pl.BlockSpec((bn, D), lambda i: (i, 0)),
            pl.BlockSpec((2, bn, D), lambda i: (0, i, 0)),
            pl.BlockSpec((D, D), lambda i: (0, 0)),
            pl.BlockSpec((D, D), lambda i: (0, 0)),
            pl.BlockSpec((1, D), lambda i: (0, 0)),
        ],
        out_specs=pl.BlockSpec((bn, D), lambda i: (i, 0)),
        out_shape=jax.ShapeDtypeStruct((N, D), jnp.float32),
    )(x, parts, wax, wah, ba)


def kernel(x, edge_attr, W1, b1, Wm1, bm1, Wm2, bm2, Wm3, bm3, Wa, ba,
           edge_index):
    u = edge_index[0, 0::2].astype(jnp.int32)
    v = edge_index[1, 0::2].astype(jnp.int32)
    uv = jnp.concatenate([u, v])
    vu = jnp.concatenate([v, u])
    ea2 = jnp.stack([edge_attr[0::2], edge_attr[1::2]])
    zeros = jnp.zeros((NP, D), jnp.float32)

    q = _tc_matmul(x, W1[:D])
    g2 = _sc_gather(q, uv).reshape(2, EH, D)
    h = _tc_lin1(ea2, g2, W1[D:], b1.reshape(1, D))

    for w, b in ((Wm1, bm1), (Wm2, bm2), (Wm3, bm3)):
        parts = _sc_scatter_add(h, vu, zeros)
        agg = _tc_sum2(parts)
        g2 = _sc_gather(agg, uv).reshape(2, EH, D)
        h = _tc_combine(h, g2, w, b.reshape(1, D))

    parts = _sc_scatter_add(h, vu, zeros)
    return _tc_final(x, parts, Wa[:D], Wa[D:], ba.reshape(1, D))


# speedup vs baseline: 3.0497x; 1.2975x over previous
import functools

import jax
import jax.numpy as jnp
from jax import lax
from jax.experimental import pallas as pl
from jax.experimental.pallas import tpu as pltpu
from jax.experimental.pallas import tpu_sc as plsc

N = 10000
EH = 160000
E = 2 * EH
D = 128

NC = 2
NS = 16
NW = NC * NS
CH = 128
NCHUNK = E // CH
CPW = 80
NCPAD = NW * CPW
NBUF = 4
SNBUF = 2
NP = 10240
RPT = NP // NS

_mesh = plsc.VectorSubcoreMesh(core_axis_name="c", subcore_axis_name="s")


def _worker_span():
    c = lax.axis_index("c")
    s = lax.axis_index("s")
    wid = s * NC + c
    base = wid * CPW
    cnt = jnp.clip(NCHUNK - base, 0, CPW)
    return base, cnt


def _sc_scatter_add(h2, idx2, zeros):
    @functools.partial(
        pl.kernel,
        out_type=jax.ShapeDtypeStruct((NC, NP, D), jnp.float32),
        mesh=_mesh,
        scratch_types=[
            pltpu.VMEM((CPW, CH), jnp.int32),
            pltpu.VMEM((SNBUF, CH, D), jnp.float32),
            pltpu.VMEM_SHARED((NP, D), jnp.float32),
        ] + [pltpu.SemaphoreType.DMA] * SNBUF,
    )
    def k(h_hbm, idx_hbm, z_hbm, out_hbm, idx_v, rows_v, acc, *sems):
        s = lax.axis_index("s")
        base, cnt = _worker_span()
        pltpu.sync_copy(z_hbm.at[pl.ds(s * RPT, RPT)], acc.at[pl.ds(s * RPT, RPT)])
        pltpu.sync_copy(idx_hbm.at[pl.ds(base, CPW)], idx_v)
        plsc.subcore_barrier()

        def load(i, b):
            return pltpu.make_async_copy(
                h_hbm.at[pl.ds((base + i) * CH, CH)], rows_v.at[b], sems[b])

        def drain(b):
            pltpu.make_async_copy(
                h_hbm.at[pl.ds(0, CH)], rows_v.at[b], sems[b]).wait()

        for b in range(SNBUF):
            @pl.when(b < cnt)
            def _(b=b):
                load(b, b).start()

        @pl.loop(0, CPW, step=SNBUF)
        def _(i0):
            for b in range(SNBUF):
                i = i0 + b

                @pl.when(i < cnt)
                def _(i=i, b=b):
                    drain(b)
                    pltpu.sync_copy(rows_v.at[b], acc.at[idx_v.at[i]], add=True)

                    @pl.when(i + SNBUF < cnt)
                    def _():
                        load(i + SNBUF, b).start()

        plsc.subcore_barrier()
        c = lax.axis_index("c")
        pltpu.sync_copy(acc.at[pl.ds(s * RPT, RPT)],
                        out_hbm.at[c, pl.ds(s * RPT, RPT)])

    return k(h2.reshape(E, D), idx2, zeros)


def _sc_gather(table, idx2):
    @functools.partial(
        pl.kernel,
        out_type=jax.ShapeDtypeStruct((E, D), jnp.float32),
        mesh=_mesh,
        scratch_types=[
            pltpu.VMEM((CPW, CH), jnp.int32),
            pltpu.VMEM((NBUF, CH, D), jnp.float32),
        ] + [pltpu.SemaphoreType.DMA] * NBUF,
    )
    def k(t_hbm, idx_hbm, out_hbm, idx_v, rows_v, *sems):
        base, cnt = _worker_span()
        pltpu.sync_copy(idx_hbm.at[pl.ds(base, CPW)], idx_v)

        def gat(i, b):
            return pltpu.make_async_copy(
                t_hbm.at[idx_v.at[i]], rows_v.at[b], sems[b])

        def drain(b):
            pltpu.make_async_copy(
                t_hbm.at[pl.ds(0, CH)], rows_v.at[b], sems[b]).wait()

        for b in range(NBUF):
            @pl.when(b < cnt)
            def _(b=b):
                gat(b, b).start()

        @pl.loop(0, CPW, step=NBUF)
        def _(i0):
            for b in range(NBUF):
                i = i0 + b

                @pl.when(i < cnt)
                def _(i=i, b=b):
                    drain(b)
                    pltpu.sync_copy(rows_v.at[b],
                                    out_hbm.at[pl.ds((base + i) * CH, CH)])

                    @pl.when(i + NBUF < cnt)
                    def _():
                        gat(i + NBUF, b).start()

    return k(table, idx2)


def _matmul_body(x_ref, w_ref, o_ref):
    o_ref[...] = jnp.dot(x_ref[...], w_ref[...],
                         preferred_element_type=jnp.float32)


def _tc_matmul(x, w, bm=2000):
    m, kdim = x.shape
    dout = w.shape[1]
    return pl.pallas_call(
        _matmul_body,
        grid=(m // bm,),
        in_specs=[
            pl.BlockSpec((bm, kdim), lambda i: (i, 0)),
            pl.BlockSpec((kdim, dout), lambda i: (0, 0)),
        ],
        out_specs=pl.BlockSpec((bm, dout), lambda i: (i, 0)),
        out_shape=jax.ShapeDtypeStruct((m, dout), jnp.float32),
    )(x, w)


def _lin1_body(ea_ref, g_ref, w_ref, b_ref, o_ref):
    w = w_ref[...]
    b = b_ref[...]
    for p in range(2):
        o_ref[p] = jnp.maximum(
            g_ref[p] + jnp.dot(ea_ref[p], w, preferred_element_type=jnp.float32)
            + b, 0.0)


def _tc_lin1(ea2, g2, w1e, b1, bp=2000):
    de = ea2.shape[-1]
    return pl.pallas_call(
        _lin1_body,
        grid=(EH // bp,),
        in_specs=[
            pl.BlockSpec((2, bp, de), lambda i: (0, i, 0)),
            pl.BlockSpec((2, bp, D), lambda i: (0, i, 0)),
            pl.BlockSpec((de, D), lambda i: (0, 0)),
            pl.BlockSpec((1, D), lambda i: (0, 0)),
        ],
        out_specs=pl.BlockSpec((2, bp, D), lambda i: (0, i, 0)),
        out_shape=jax.ShapeDtypeStruct((2, EH, D), jnp.float32),
    )(ea2, g2, w1e, b1)


def _combine_body(h_ref, g_ref, w_ref, b_ref, o_ref):
    w = w_ref[...]
    b = b_ref[...]
    he = h_ref[0]
    ho = h_ref[1]
    o_ref[0] = jnp.maximum(
        he + jnp.dot(g_ref[0] - ho, w, preferred_element_type=jnp.float32) + b,
        0.0)
    o_ref[1] = jnp.maximum(
        ho + jnp.dot(g_ref[1] - he, w, preferred_element_type=jnp.float32) + b,
        0.0)


def _tc_combine(h2, g2, w, b, bp=2000):
    return pl.pallas_call(
        _combine_body,
        grid=(EH // bp,),
        in_specs=[
            pl.BlockSpec((2, bp, D), lambda i: (0, i, 0)),
            pl.BlockSpec((2, bp, D), lambda i: (0, i, 0)),
            pl.BlockSpec((D, D), lambda i: (0, 0)),
            pl.BlockSpec((1, D), lambda i: (0, 0)),
        ],
        out_specs=pl.BlockSpec((2, bp, D), lambda i: (0, i, 0)),
        out_shape=jax.ShapeDtypeStruct((2, EH, D), jnp.float32),
    )(h2, g2, w, b)


def _sum2_body(p_ref, o_ref):
    o_ref[...] = p_ref[0] + p_ref[1]


def _tc_sum2(parts, bn=2000):
    return pl.pallas_call(
        _sum2_body,
        grid=(N // bn,),
        in_specs=[pl.BlockSpec((2, bn, D), lambda i: (0, i, 0))],
        out_specs=pl.BlockSpec((bn, D), lambda i: (i, 0)),
        out_shape=jax.ShapeDtypeStruct((N, D), jnp.float32),
    )(parts)


def _final_body(x_ref, p_ref, wx_ref, wh_ref, b_ref, o_ref):
    agg = p_ref[0] + p_ref[1]
    o_ref[...] = jnp.maximum(
        jnp.dot(x_ref[...], wx_ref[...], preferred_element_type=jnp.float32)
        + jnp.dot(agg, wh_ref[...], preferred_element_type=jnp.float32)
        + b_ref[...], 0.0)


def _tc_final(x, parts, wax, wah, ba, bn=2000):
    return pl.pallas_call(
        _final_body,
        grid=(N // bn,),
        in_specs=[
            pl.BlockSpec((bn, D), lambda i: (i, 0)),
            pl.BlockSpec((2, bn, D), lambda i: (0, i, 0)),
            pl.BlockSpec((D, D), lambda i: (0, 0)),
            pl.BlockSpec((D, D), lambda i: (0, 0)),
            pl.BlockSpec((1, D), lambda i: (0, 0)),
        ],
        out_specs=pl.BlockSpec((bn, D), lambda i: (i, 0)),
        out_shape=jax.ShapeDtypeStruct((N, D), jnp.float32),
    )(x, parts, wax, wah, ba)


def kernel(x, edge_attr, W1, b1, Wm1, bm1, Wm2, bm2, Wm3, bm3, Wa, ba,
           edge_index):
    u = edge_index[0, 0::2].astype(jnp.int32)
    v = edge_index[1, 0::2].astype(jnp.int32)
    pad = jnp.zeros((NCPAD * CH - E,), jnp.int32)
    uv = jnp.concatenate([u, v, pad]).reshape(NCPAD, CH)
    vu = jnp.concatenate([v, u, pad]).reshape(NCPAD, CH)
    ea2 = jnp.stack([edge_attr[0::2], edge_attr[1::2]])
    zeros = jnp.zeros((NP, D), jnp.float32)

    q = _tc_matmul(x, W1[:D])
    g2 = _sc_gather(q, uv).reshape(2, EH, D)
    h = _tc_lin1(ea2, g2, W1[D:], b1.reshape(1, D))

    for w, b in ((Wm1, bm1), (Wm2, bm2), (Wm3, bm3)):
        parts = _sc_scatter_add(h, vu, zeros)
        agg = _tc_sum2(parts)
        g2 = _sc_gather(agg, uv).reshape(2, EH, D)
        h = _tc_combine(h, g2, w, b.reshape(1, D))

    parts = _sc_scatter_add(h, vu, zeros)
    return _tc_final(x, parts, Wa[:D], Wa[D:], ba.reshape(1, D))
